# baseline (speedup 1.0000x reference)
import jax
import jax.numpy as jnp
import numpy as np
from jax import lax
from jax.experimental import pallas as pl
from jax.experimental.pallas import tpu as pltpu
from jax.experimental.pallas import tpu_sc as plsc

L = 16
FPL = 2
T = 2 ** 19
N_MIN = 16
SCALE = 2.0
B_GROWTH = float(np.exp(np.log(2048.0 * SCALE / N_MIN) / (L - 1)))
P1 = int(np.uint32(2654435761).astype(np.int32))
P2 = int(np.uint32(805459861).astype(np.int32))
N_PTS = 524288
D_HID = 64
D_OUT = 96
D_ENC = L * FPL

_SCALES = np.array([N_MIN * B_GROWTH ** l - 1.0 for l in range(L)], np.float32)
_RES = np.array([int(np.ceil(s)) + 1 for s in _SCALES], np.int32)
_DENSE = np.array([int(int(r) ** 3 <= T) for r in _RES], np.int32)
_RES2 = (_RES.astype(np.int64) ** 2).astype(np.int32)
_LBASE = (np.arange(L, dtype=np.int64) * (T // 4)).astype(np.int32)
N_DENSE = int(_DENSE.sum())

NC, NS = 2, 16
NW = NC * NS
C = 512
PPW = N_PTS // NW
NCHUNK = PPW // C
IDX_PER_DMA = 128
G = 8 * C // IDX_PER_DMA


def _splat(ref, lv):
    return plsc.load_gather(ref, [jnp.full((16,), lv, jnp.int32)])


def _pos_frac(xbuf, s, scale, lanes):
    pi, fr, om = [], [], []
    row = lanes + s
    for d in range(3):
        v = plsc.load_gather(xbuf, [row, jnp.full((16,), d, jnp.int32)])
        xn = jnp.minimum(jnp.maximum(v * 0.5 + 0.5, 0.0), 1.0)
        pos = xn * scale + 0.5
        p0 = pos.astype(jnp.int32)
        f = pos - p0.astype(jnp.float32)
        pi.append(p0)
        fr.append(f)
        om.append(1.0 - f)
    return pi, fr, om


def _weights(fr, om):
    wx = (om[0], fr[0])
    wy = (om[1], fr[1])
    wz = (om[2], fr[2])
    wxy = [wx[cx] * wy[cy] for cy in (0, 1) for cx in (0, 1)]
    return wxy, wz


def _store_corner(idxbuf, obuf, wbuf, c, s, idx, w):
    p = c * C + s
    idxbuf[p // IDX_PER_DMA, pl.ds(p % IDX_PER_DMA, 16)] = (
        lax.shift_right_logical(idx, 2))
    obuf[pl.ds(p, 16)] = lax.shift_left(lax.bitwise_and(idx, 3), 1)
    wbuf[pl.ds(p, 16)] = w


def _fire_drain_gathers(tbl, idxbuf, rowsbuf, sem):
    descs = [
        pltpu.async_copy(tbl.at[idxbuf.at[gg]],
                         rowsbuf.at[pl.ds(gg * IDX_PER_DMA, IDX_PER_DMA), :],
                         sem)
        for gg in range(G)
    ]
    for d in descs:
        d.wait()


def _accumulate(lv, obuf, wbuf, rowsbuf, featbuf, lanes):
    @pl.loop(0, C // 16)
    def _acc(g):
        p0 = g * 16
        acc0 = jnp.zeros((16,), jnp.float32)
        acc1 = jnp.zeros((16,), jnp.float32)
        for c in range(8):
            cbase = c * C + p0
            o16 = obuf[pl.ds(cbase, 16)]
            w16 = wbuf[pl.ds(cbase, 16)]
            i16 = lanes + cbase
            r0 = plsc.load_gather(rowsbuf, [i16, o16])
            r1 = plsc.load_gather(rowsbuf, [i16, o16 + 1])
            acc0 = acc0 + w16 * r0
            acc1 = acc1 + w16 * r1
        s0 = lanes * D_ENC + (p0 * D_ENC + 2 * lv)
        plsc.store_scatter(featbuf, [s0], acc0)
        plsc.store_scatter(featbuf, [s0 + 1], acc1)


def _sc_encode_body(xt, tbl, scalef, resi, res2i, lbasei, enc,
                    pscale, pres, pres2, plbase,
                    xbuf, idxbuf, obuf, wbuf, rowsbuf, featbuf, sem):
    wid = lax.axis_index("s") * NC + lax.axis_index("c")

    pltpu.sync_copy(scalef, pscale)
    pltpu.sync_copy(resi, pres)
    pltpu.sync_copy(res2i, pres2)
    pltpu.sync_copy(lbasei, plbase)

    lanes = lax.iota(jnp.int32, 16)

    @pl.loop(0, NCHUNK)
    def _chunk(ci):
        base = wid * PPW + ci * C
        pltpu.sync_copy(xt.at[pl.ds(base, C), :], xbuf)

        @pl.loop(0, N_DENSE)
        def _dense_level(lv):
            scale = _splat(pscale, lv)
            resv = _splat(pres, lv)
            res2v = _splat(pres2, lv)
            lbase = _splat(plbase, lv)
            resm1 = resv - 1

            @pl.loop(0, C // 16)
            def _idxw(g):
                s = g * 16
                pi, fr, om = _pos_frac(xbuf, s, scale, lanes)
                wxy, wz = _weights(fr, om)
                da = (jnp.minimum(pi[0], resm1), jnp.minimum(pi[0] + 1, resm1))
                db0 = jnp.minimum(pi[1], resm1)
                db1 = jnp.minimum(pi[1] + 1, resm1)
                db = (db0 * resv, db1 * resv)
                dc0 = jnp.minimum(pi[2], resm1)
                dc1 = jnp.minimum(pi[2] + 1, resm1)
                dc = (dc0 * res2v, dc1 * res2v)
                for c in range(8):
                    bx, by, bz = c & 1, (c >> 1) & 1, (c >> 2) & 1
                    idx = da[bx] + db[by] + dc[bz]
                    w = wxy[c & 3] * wz[bz]
                    p = c * C + s
                    idxbuf[p // IDX_PER_DMA, pl.ds(p % IDX_PER_DMA, 16)] = (
                        lax.shift_right_logical(idx, 2) + lbase)
                    obuf[pl.ds(p, 16)] = lax.shift_left(
                        lax.bitwise_and(idx, 3), 1)
                    wbuf[pl.ds(p, 16)] = w

            _fire_drain_gathers(tbl, idxbuf, rowsbuf, sem)
            _accumulate(lv, obuf, wbuf, rowsbuf, featbuf, lanes)

        @pl.loop(N_DENSE, L)
        def _hash_level(lv):
            scale = _splat(pscale, lv)
            lbase = _splat(plbase, lv)

            @pl.loop(0, C // 16)
            def _idxw(g):
                s = g * 16
                pi, fr, om = _pos_frac(xbuf, s, scale, lanes)
                wxy, wz = _weights(fr, om)
                ha = (pi[0], pi[0] + 1)
                hb0 = pi[1] * P1
                hb = (hb0, hb0 + P1)
                hc0 = pi[2] * P2
                hc = (hc0, hc0 + P2)
                for c in range(8):
                    bx, by, bz = c & 1, (c >> 1) & 1, (c >> 2) & 1
                    idx = lax.bitwise_and(
                        lax.bitwise_xor(lax.bitwise_xor(ha[bx], hb[by]),
                                        hc[bz]),
                        T - 1)
                    w = wxy[c & 3] * wz[bz]
                    p = c * C + s
                    idxbuf[p // IDX_PER_DMA, pl.ds(p % IDX_PER_DMA, 16)] = (
                        lax.shift_right_logical(idx, 2) + lbase)
                    obuf[pl.ds(p, 16)] = lax.shift_left(
                        lax.bitwise_and(idx, 3), 1)
                    wbuf[pl.ds(p, 16)] = w

            _fire_drain_gathers(tbl, idxbuf, rowsbuf, sem)
            _accumulate(lv, obuf, wbuf, rowsbuf, featbuf, lanes)

        pltpu.sync_copy(featbuf, enc.at[pl.ds(base * D_ENC, C * D_ENC)])


def _sc_encode(xt, tbl, scalef, resi, res2i, lbasei):
    mesh = plsc.VectorSubcoreMesh(core_axis_name="c", subcore_axis_name="s",
                                  num_cores=NC, num_subcores=NS)
    fn = pl.kernel(
        _sc_encode_body,
        out_type=jax.ShapeDtypeStruct((N_PTS * D_ENC,), jnp.float32),
        mesh=mesh,
        compiler_params=pltpu.CompilerParams(needs_layout_passes=False,
                                             use_tc_tiling_on_sc=False),
        scratch_types=[
            pltpu.VMEM((L,), jnp.float32),
            pltpu.VMEM((L,), jnp.int32),
            pltpu.VMEM((L,), jnp.int32),
            pltpu.VMEM((L,), jnp.int32),
            pltpu.VMEM((C, 3), jnp.float32),
            pltpu.VMEM((G, IDX_PER_DMA), jnp.int32),
            pltpu.VMEM((8 * C,), jnp.int32),
            pltpu.VMEM((8 * C,), jnp.float32),
            pltpu.VMEM((8 * C, 8), jnp.float32),
            pltpu.VMEM((C * D_ENC,), jnp.float32),
            pltpu.SemaphoreType.DMA,
        ],
    )
    return fn(xt, tbl, scalef, resi, res2i, lbasei)


def _mlp_body(enc_ref, w1_ref, w2_ref, out_ref):
    h = jnp.maximum(
        jnp.dot(enc_ref[...], w1_ref[...],
                preferred_element_type=jnp.float32), 0.0)
    out_ref[...] = jnp.maximum(
        jnp.dot(h, w2_ref[...], preferred_element_type=jnp.float32), 0.0)


def _mlp(enc, W1, W2):
    BM = 4096
    return pl.pallas_call(
        _mlp_body,
        grid=(N_PTS // BM,),
        in_specs=[
            pl.BlockSpec((BM, D_ENC), lambda i: (i, 0)),
            pl.BlockSpec((D_ENC, D_HID), lambda i: (0, 0)),
            pl.BlockSpec((D_HID, D_OUT), lambda i: (0, 0)),
        ],
        out_specs=pl.BlockSpec((BM, D_OUT), lambda i: (i, 0)),
        out_shape=jax.ShapeDtypeStruct((N_PTS, D_OUT), jnp.float32),
    )(enc, W1, W2)


@jax.jit
def kernel(x, table, W1, W2):
    tbl = table.reshape(L * T * FPL // 8, 8)
    enc_flat = _sc_encode(x, tbl,
                          jnp.asarray(_SCALES), jnp.asarray(_RES),
                          jnp.asarray(_RES2), jnp.asarray(_LBASE))
    return _mlp(enc_flat.reshape(N_PTS, D_ENC), W1, W2)


# speedup vs baseline: 1.0681x; 1.0681x over previous
import jax
import jax.numpy as jnp
import numpy as np
from jax import lax
from jax.experimental import pallas as pl
from jax.experimental.pallas import tpu as pltpu
from jax.experimental.pallas import tpu_sc as plsc

L = 16
FPL = 2
T = 2 ** 19
N_MIN = 16
SCALE = 2.0
B_GROWTH = float(np.exp(np.log(2048.0 * SCALE / N_MIN) / (L - 1)))
P1 = int(np.uint32(2654435761).astype(np.int32))
P2 = int(np.uint32(805459861).astype(np.int32))
N_PTS = 524288
D_HID = 64
D_OUT = 96
D_ENC = L * FPL

_SCALES = np.array([N_MIN * B_GROWTH ** l - 1.0 for l in range(L)], np.float32)
_RES = np.array([int(np.ceil(s)) + 1 for s in _SCALES], np.int32)
_DENSE = np.array([int(int(r) ** 3 <= T) for r in _RES], np.int32)
_RES2 = (_RES.astype(np.int64) ** 2).astype(np.int32)
_LBASE = (np.arange(L, dtype=np.int64) * (T // 4)).astype(np.int32)
N_DENSE = int(_DENSE.sum())

NC, NS = 2, 16
NW = NC * NS
C = 512
PPW = N_PTS // NW
NCHUNK = PPW // C
IDX_PER_DMA = 128
G = 8 * C // IDX_PER_DMA


def _splat(ref, lv):
    return plsc.load_gather(ref, [jnp.full((16,), lv, jnp.int32)])


def _pos_frac(xbuf, s, scale):
    pi, fr, om = [], [], []
    for d in range(3):
        v = xbuf[d, pl.ds(s, 16)]
        xn = jnp.minimum(jnp.maximum(v * 0.5 + 0.5, 0.0), 1.0)
        pos = xn * scale + 0.5
        p0 = pos.astype(jnp.int32)
        f = pos - p0.astype(jnp.float32)
        pi.append(p0)
        fr.append(f)
        om.append(1.0 - f)
    return pi, fr, om


def _weights(fr, om):
    wx = (om[0], fr[0])
    wy = (om[1], fr[1])
    wz = (om[2], fr[2])
    wxy = [wx[cx] * wy[cy] for cy in (0, 1) for cx in (0, 1)]
    return wxy, wz


def _store_corner(idxbuf, obuf, wbuf, c, s, idx, w):
    p = c * C + s
    idxbuf[p // IDX_PER_DMA, pl.ds(p % IDX_PER_DMA, 16)] = (
        lax.shift_right_logical(idx, 2))
    obuf[pl.ds(p, 16)] = lax.shift_left(lax.bitwise_and(idx, 3), 1)
    wbuf[pl.ds(p, 16)] = w


def _fire_drain_gathers(tbl, idxbuf, rowsbuf, sem):
    descs = [
        pltpu.async_copy(tbl.at[idxbuf.at[gg]],
                         rowsbuf.at[pl.ds(gg * IDX_PER_DMA, IDX_PER_DMA), :],
                         sem)
        for gg in range(G)
    ]
    for d in descs:
        d.wait()


def _accumulate(lv, obuf, wbuf, rowsbuf, featbuf, lanes):
    @pl.loop(0, C // 16)
    def _acc(g):
        p0 = g * 16
        acc0 = jnp.zeros((16,), jnp.float32)
        acc1 = jnp.zeros((16,), jnp.float32)
        for c in range(8):
            cbase = c * C + p0
            o16 = obuf[pl.ds(cbase, 16)]
            w16 = wbuf[pl.ds(cbase, 16)]
            i16 = lanes + cbase
            r0 = plsc.load_gather(rowsbuf, [i16, o16])
            r1 = plsc.load_gather(rowsbuf, [i16, o16 + 1])
            acc0 = acc0 + w16 * r0
            acc1 = acc1 + w16 * r1
        featbuf[2 * lv, pl.ds(p0, 16)] = acc0
        featbuf[2 * lv + 1, pl.ds(p0, 16)] = acc1


def _sc_encode_body(xt, tbl, scalef, resi, res2i, lbasei, enc,
                    pscale, pres, pres2, plbase,
                    xbuf, idxbuf, obuf, wbuf, rowsbuf, featbuf, sem):
    wid = lax.axis_index("s") * NC + lax.axis_index("c")

    pltpu.sync_copy(scalef, pscale)
    pltpu.sync_copy(resi, pres)
    pltpu.sync_copy(res2i, pres2)
    pltpu.sync_copy(lbasei, plbase)

    lanes = lax.iota(jnp.int32, 16)

    @pl.loop(0, NCHUNK)
    def _chunk(ci):
        base = wid * PPW + ci * C
        pltpu.sync_copy(xt.at[:, pl.ds(base, C)], xbuf)

        @pl.loop(0, N_DENSE)
        def _dense_level(lv):
            scale = _splat(pscale, lv)
            resv = _splat(pres, lv)
            res2v = _splat(pres2, lv)
            lbase = _splat(plbase, lv)
            resm1 = resv - 1

            @pl.loop(0, C // 16)
            def _idxw(g):
                s = g * 16
                pi, fr, om = _pos_frac(xbuf, s, scale)
                wxy, wz = _weights(fr, om)
                da = (jnp.minimum(pi[0], resm1), jnp.minimum(pi[0] + 1, resm1))
                db0 = jnp.minimum(pi[1], resm1)
                db1 = jnp.minimum(pi[1] + 1, resm1)
                db = (db0 * resv, db1 * resv)
                dc0 = jnp.minimum(pi[2], resm1)
                dc1 = jnp.minimum(pi[2] + 1, resm1)
                dc = (dc0 * res2v, dc1 * res2v)
                for c in range(8):
                    bx, by, bz = c & 1, (c >> 1) & 1, (c >> 2) & 1
                    idx = da[bx] + db[by] + dc[bz]
                    w = wxy[c & 3] * wz[bz]
                    p = c * C + s
                    idxbuf[p // IDX_PER_DMA, pl.ds(p % IDX_PER_DMA, 16)] = (
                        lax.shift_right_logical(idx, 2) + lbase)
                    obuf[pl.ds(p, 16)] = lax.shift_left(
                        lax.bitwise_and(idx, 3), 1)
                    wbuf[pl.ds(p, 16)] = w

            _fire_drain_gathers(tbl, idxbuf, rowsbuf, sem)
            _accumulate(lv, obuf, wbuf, rowsbuf, featbuf, lanes)

        @pl.loop(N_DENSE, L)
        def _hash_level(lv):
            scale = _splat(pscale, lv)
            lbase = _splat(plbase, lv)

            @pl.loop(0, C // 16)
            def _idxw(g):
                s = g * 16
                pi, fr, om = _pos_frac(xbuf, s, scale)
                wxy, wz = _weights(fr, om)
                ha = (pi[0], pi[0] + 1)
                hb0 = pi[1] * P1
                hb = (hb0, hb0 + P1)
                hc0 = pi[2] * P2
                hc = (hc0, hc0 + P2)
                for c in range(8):
                    bx, by, bz = c & 1, (c >> 1) & 1, (c >> 2) & 1
                    idx = lax.bitwise_and(
                        lax.bitwise_xor(lax.bitwise_xor(ha[bx], hb[by]),
                                        hc[bz]),
                        T - 1)
                    w = wxy[c & 3] * wz[bz]
                    p = c * C + s
                    idxbuf[p // IDX_PER_DMA, pl.ds(p % IDX_PER_DMA, 16)] = (
                        lax.shift_right_logical(idx, 2) + lbase)
                    obuf[pl.ds(p, 16)] = lax.shift_left(
                        lax.bitwise_and(idx, 3), 1)
                    wbuf[pl.ds(p, 16)] = w

            _fire_drain_gathers(tbl, idxbuf, rowsbuf, sem)
            _accumulate(lv, obuf, wbuf, rowsbuf, featbuf, lanes)

        pltpu.sync_copy(featbuf, enc.at[:, pl.ds(base, C)])


def _sc_encode(xt, tbl, scalef, resi, res2i, lbasei):
    mesh = plsc.VectorSubcoreMesh(core_axis_name="c", subcore_axis_name="s",
                                  num_cores=NC, num_subcores=NS)
    fn = pl.kernel(
        _sc_encode_body,
        out_type=jax.ShapeDtypeStruct((D_ENC, N_PTS), jnp.float32),
        mesh=mesh,
        compiler_params=pltpu.CompilerParams(needs_layout_passes=False,
                                             use_tc_tiling_on_sc=False),
        scratch_types=[
            pltpu.VMEM((L,), jnp.float32),
            pltpu.VMEM((L,), jnp.int32),
            pltpu.VMEM((L,), jnp.int32),
            pltpu.VMEM((L,), jnp.int32),
            pltpu.VMEM((3, C), jnp.float32),
            pltpu.VMEM((G, IDX_PER_DMA), jnp.int32),
            pltpu.VMEM((8 * C,), jnp.int32),
            pltpu.VMEM((8 * C,), jnp.float32),
            pltpu.VMEM((8 * C, 8), jnp.float32),
            pltpu.VMEM((D_ENC, C), jnp.float32),
            pltpu.SemaphoreType.DMA,
        ],
    )
    return fn(xt, tbl, scalef, resi, res2i, lbasei)


def _mlp_body(enc_ref, w1_ref, w2_ref, out_ref):
    h = jnp.maximum(
        lax.dot_general(enc_ref[...], w1_ref[...], (((0,), (0,)), ((), ())),
                        preferred_element_type=jnp.float32), 0.0)
    out_ref[...] = jnp.maximum(
        jnp.dot(h, w2_ref[...], preferred_element_type=jnp.float32), 0.0)


def _mlp(enc, W1, W2):
    BM = 4096
    return pl.pallas_call(
        _mlp_body,
        grid=(N_PTS // BM,),
        in_specs=[
            pl.BlockSpec((D_ENC, BM), lambda i: (0, i)),
            pl.BlockSpec((D_ENC, D_HID), lambda i: (0, 0)),
            pl.BlockSpec((D_HID, D_OUT), lambda i: (0, 0)),
        ],
        out_specs=pl.BlockSpec((BM, D_OUT), lambda i: (i, 0)),
        out_shape=jax.ShapeDtypeStruct((N_PTS, D_OUT), jnp.float32),
    )(enc, W1, W2)


@jax.jit
def kernel(x, table, W1, W2):
    xt = x.T
    tbl = table.reshape(L * T * FPL // 8, 8)
    enc_t = _sc_encode(xt, tbl,
                       jnp.asarray(_SCALES), jnp.asarray(_RES),
                       jnp.asarray(_RES2), jnp.asarray(_LBASE))
    return _mlp(enc_t, W1, W2)
